# Initial kernel scaffold; baseline (speedup 1.0000x reference)
import jax
import jax.numpy as jnp
from jax.experimental import pallas as pl


def kernel(inputs, w_sparse, w_dense, W1, b1, W2, b2, W3, b3, Wp, bp):
    raise NotImplementedError("write your pallas kernel here")


# speedup vs baseline: 15.6269x; 15.6269x over previous
import functools

import jax
import jax.numpy as jnp
from jax import lax
from jax.experimental import pallas as pl
from jax.experimental.pallas import tpu as pltpu
from jax.experimental.pallas import tpu_sc as plsc

B, F, V, D, H = 16384, 26, 1000000, 16, 100
BF = B * F
NW = 32
CHUNK = BF // NW
NSUB = 4
SUB = CHUNK // NSUB


def _sc_gather(idx_flat, idx_rows, w_dense, w_sparse16):
    mesh = plsc.VectorSubcoreMesh(core_axis_name="c", subcore_axis_name="s")

    @functools.partial(
        pl.kernel,
        out_type=(
            jax.ShapeDtypeStruct((BF, D), jnp.float32),
            jax.ShapeDtypeStruct((BF,), jnp.float32),
        ),
        mesh=mesh,
        compiler_params=pltpu.CompilerParams(use_tc_tiling_on_sc=False,
                                             needs_layout_passes=False),
        scratch_types=[
            pltpu.VMEM((SUB,), jnp.int32),
            pltpu.VMEM((SUB,), jnp.int32),
            pltpu.VMEM((SUB, D), jnp.float32),
            pltpu.VMEM((SUB, 16), jnp.float32),
            pltpu.VMEM((SUB,), jnp.float32),
            pltpu.SemaphoreType.DMA,
            pltpu.SemaphoreType.DMA,
        ],
    )
    def k(idx_hbm, idxr_hbm, wd_hbm, ws_hbm, dense_out, wide_out,
          idx_v, idxr_v, rows_v, srows_v, sv_v, sem1, sem2):
        wid = lax.axis_index("s") * 2 + lax.axis_index("c")
        lane = lax.iota(jnp.int32, 16)
        for j in range(NSUB):
            base = wid * CHUNK + j * SUB
            pltpu.sync_copy(idx_hbm.at[pl.ds(base, SUB)], idx_v)
            pltpu.sync_copy(idxr_hbm.at[pl.ds(base, SUB)], idxr_v)
            cp1 = pltpu.async_copy(wd_hbm.at[idx_v], rows_v, sem1)
            cp2 = pltpu.async_copy(ws_hbm.at[idxr_v], srows_v, sem2)
            cp1.wait()
            cp2.wait()

            def extract(kk, carry):
                off = kk * 16
                col = idx_v[pl.ds(off, 16)] & 15
                row = lane + off
                sv_v[pl.ds(off, 16)] = plsc.load_gather(srows_v, [row, col])
                return carry

            lax.fori_loop(0, SUB // 16, extract, 0)
            pltpu.sync_copy(rows_v, dense_out.at[pl.ds(base, SUB)])
            pltpu.sync_copy(sv_v, wide_out.at[pl.ds(base, SUB)])

    return k(idx_flat, idx_rows, w_dense, w_sparse16)


def _mlp_body(x_ref, sv_ref, w1_ref, b1_ref, w2_ref, b2_ref, w3_ref, b3_ref,
              wp_ref, bp_ref, out_ref):
    x = x_ref[...]
    h = jnp.dot(x, w1_ref[...], preferred_element_type=jnp.float32) + b1_ref[...]
    h = h * jax.nn.sigmoid(h)
    h = jnp.dot(h, w2_ref[...], preferred_element_type=jnp.float32) + b2_ref[...]
    h = h * jax.nn.sigmoid(h)
    h = jnp.dot(h, w3_ref[...], preferred_element_type=jnp.float32) + b3_ref[...]
    h = h * jax.nn.sigmoid(h)
    logits = jnp.sum(h * wp_ref[...], axis=1, keepdims=True) + bp_ref[0, 0]
    logits = logits + jnp.sum(sv_ref[...], axis=1, keepdims=True)
    out_ref[...] = jax.nn.sigmoid(logits)


def _tc_mlp(x, sv, W1p, b1p, W2p, b2p, W3p, b3p, wp_row, bp):
    BM = 2048
    grid = (B // BM,)
    full = lambda shape: pl.BlockSpec(shape, lambda i: (0, 0))
    return pl.pallas_call(
        _mlp_body,
        grid=grid,
        in_specs=[
            pl.BlockSpec((BM, F * D), lambda i: (i, 0)),
            pl.BlockSpec((BM, F), lambda i: (i, 0)),
            full((F * D, 128)),
            full((1, 128)),
            full((128, 128)),
            full((1, 128)),
            full((128, 128)),
            full((1, 128)),
            full((1, 128)),
            full((1, 1)),
        ],
        out_specs=pl.BlockSpec((BM, 1), lambda i: (i, 0)),
        out_shape=jax.ShapeDtypeStruct((B, 1), jnp.float32),
    )(x, sv, W1p, b1p, W2p, b2p, W3p, b3p, wp_row, bp)


def kernel(inputs, w_sparse, w_dense, W1, b1, W2, b2, W3, b3, Wp, bp):
    idx_flat = inputs.reshape(-1)
    idx_rows = idx_flat >> 4
    w_sparse16 = w_sparse.reshape(V // 16, 16)
    dense_rows, wide_vals = _sc_gather(idx_flat, idx_rows, w_dense, w_sparse16)
    x = dense_rows.reshape(B, F * D)
    sv = wide_vals.reshape(B, F)

    pad = 128 - H
    W1p = jnp.pad(W1, ((0, 0), (0, pad)))
    b1p = jnp.pad(b1, (0, pad)).reshape(1, 128)
    W2p = jnp.pad(W2, ((0, pad), (0, pad)))
    b2p = jnp.pad(b2, (0, pad)).reshape(1, 128)
    W3p = jnp.pad(W3, ((0, pad), (0, pad)))
    b3p = jnp.pad(b3, (0, pad)).reshape(1, 128)
    wp_row = jnp.pad(Wp[:, 0], (0, pad)).reshape(1, 128)
    bp2 = bp.reshape(1, 1)

    return _tc_mlp(x, sv, W1p, b1p, W2p, b2p, W3p, b3p, wp_row, bp2)
